# baseline (device time: 121545 ns/iter reference)
import jax
import jax.numpy as jnp
from jax import lax
from jax.experimental import pallas as pl
from jax.experimental.pallas import tpu as pltpu

H = 16
DH = 128
DR = 32
NXY = 4
NC = 2


def kernel(x, Wdkv, Wuk, Wuv, Wq, Wqr, Wkr, Wo):
    B, S, D = x.shape
    Dc = Wdkv.shape[1]
    R = S // NXY
    RC = R // NC

    bf = jnp.bfloat16
    xb = x.astype(bf)
    wdkv = Wdkv.astype(bf)
    wuk = Wuk.astype(bf)
    wuv = Wuv.astype(bf)
    wq = Wq.astype(bf)
    wqr = Wqr.astype(bf)
    wkr = Wkr.astype(bf)
    wo = Wo.astype(bf)

    def body(x_ref, wdkv_ref, wuk_ref, wuv_ref, wq_ref, wqr_ref, wkr_ref,
             wo_ref, out_ref, c_ref, c_recv, wuk_recv, wuv_recv, oblk_ref,
             rblk, zs_sems, zr_sems, rs_sems, rr_sems):
        my_x = lax.axis_index("x")
        my_y = lax.axis_index("y")
        my_z = lax.axis_index("z")
        zpeer = (my_x, my_y, 1 - my_z)
        b = 2 * my_x + my_y
        peers = [
            (my_x, 1 - my_y, my_z),
            (1 - my_x, my_y, my_z),
            (1 - my_x, 1 - my_y, my_z),
        ]

        barrier_sem = pltpu.get_barrier_semaphore()
        for nbr in [zpeer] + peers:
            pl.semaphore_signal(barrier_sem, inc=1, device_id=nbr,
                                device_id_type=pl.DeviceIdType.MESH)
        pl.semaphore_wait(barrier_sem, 4)

        xv = x_ref[0]
        c = jnp.dot(xv, wdkv_ref[...],
                    preferred_element_type=jnp.float32).astype(bf)
        c_ref[...] = c

        zrdmas = []
        for i, (src, dst) in enumerate(
            [(c_ref, c_recv), (wuk_ref, wuk_recv), (wuv_ref, wuv_recv)]
        ):
            rdma = pltpu.make_async_remote_copy(
                src_ref=src, dst_ref=dst,
                send_sem=zs_sems.at[i], recv_sem=zr_sems.at[i],
                device_id=zpeer, device_id_type=pl.DeviceIdType.MESH,
            )
            rdma.start()
            zrdmas.append(rdma)

        xq = x_ref[0, pl.ds(b * R, R), :]
        q = jnp.dot(xq, wq_ref[...], preferred_element_type=jnp.float32
                    ).astype(bf)
        qr = jnp.dot(xq, wqr_ref[...], preferred_element_type=jnp.float32
                     ).astype(bf)
        kr = jnp.dot(xv, wkr_ref[...], preferred_element_type=jnp.float32
                     ).astype(bf)

        for rdma in zrdmas:
            rdma.wait()

        k = (jnp.dot(c, wuk_ref[...], preferred_element_type=jnp.float32)
             + jnp.dot(c_recv[...], wuk_recv[...],
                       preferred_element_type=jnp.float32)).astype(bf)
        v = (jnp.dot(c, wuv_ref[...], preferred_element_type=jnp.float32)
             + jnp.dot(c_recv[...], wuv_recv[...],
                       preferred_element_type=jnp.float32)).astype(bf)

        scale = (DH + DR) ** -0.5
        krt = kr.T
        send_rdmas = []
        for j in range(NC):
            outs = []
            for h in range(H):
                qh = q[j * RC:(j + 1) * RC, h * DH:(h + 1) * DH]
                kh = k[:, h * DH:(h + 1) * DH]
                qrh = qr[j * RC:(j + 1) * RC, h * DR:(h + 1) * DR]
                s = (jnp.dot(qh, kh.T, preferred_element_type=jnp.float32)
                     + jnp.dot(qrh, krt, preferred_element_type=jnp.float32)
                     ) * scale
                m = jnp.max(s, axis=-1, keepdims=True)
                pj = jnp.exp(s - m)
                pj = (pj / jnp.sum(pj, axis=-1, keepdims=True)).astype(bf)
                vh = v[:, h * DH:(h + 1) * DH]
                outs.append(jnp.dot(pj, vh,
                                    preferred_element_type=jnp.float32))
            o = jnp.concatenate(outs, axis=1).astype(bf)
            out_rows = jnp.dot(o, wo_ref[...],
                               preferred_element_type=jnp.float32)
            out_ref[0, pl.ds(b * R + j * RC, RC), :] = out_rows
            oblk_ref[j] = out_rows.astype(bf)
            for t, tgt in enumerate(peers):
                rdma = pltpu.make_async_remote_copy(
                    src_ref=oblk_ref.at[j], dst_ref=rblk.at[b, j],
                    send_sem=rs_sems.at[t, j], recv_sem=rr_sems.at[b, j],
                    device_id=tgt, device_id_type=pl.DeviceIdType.MESH,
                )
                rdma.start()
                send_rdmas.append(rdma)

        for src in range(NXY):
            for j in range(NC):
                @pl.when(src != b)
                def _(src=src, j=j):
                    recv = pltpu.make_async_remote_copy(
                        src_ref=oblk_ref.at[j], dst_ref=rblk.at[src, j],
                        send_sem=rs_sems.at[0, j], recv_sem=rr_sems.at[src, j],
                        device_id=zpeer, device_id_type=pl.DeviceIdType.MESH,
                    )
                    recv.wait_recv()
                    out_ref[0, pl.ds(src * R + j * RC, RC), :] = (
                        rblk[src, j].astype(jnp.float32))

        for rdma in send_rdmas:
            rdma.wait_send()

    out = pl.pallas_call(
        body,
        out_shape=jax.ShapeDtypeStruct((B, S, D), jnp.float32),
        in_specs=[pl.BlockSpec(memory_space=pltpu.VMEM)] * 8,
        out_specs=pl.BlockSpec(memory_space=pltpu.VMEM),
        scratch_shapes=[
            pltpu.VMEM((S, Dc), bf),
            pltpu.VMEM((S, Dc), bf),
            pltpu.VMEM((Dc, D), bf),
            pltpu.VMEM((Dc, D), bf),
            pltpu.VMEM((NC, RC, D), bf),
            pltpu.VMEM((NXY, NC, RC, D), bf),
            pltpu.SemaphoreType.DMA((3,)),
            pltpu.SemaphoreType.DMA((3,)),
            pltpu.SemaphoreType.DMA((3, NC)),
            pltpu.SemaphoreType.DMA((NXY, NC)),
        ],
        compiler_params=pltpu.CompilerParams(
            collective_id=0, vmem_limit_bytes=100 * 2**20
        ),
    )(xb, wdkv, wuk, wuv, wq, wqr, wkr, wo)

    return out
